# baseline (speedup 1.0000x reference)
import math

import jax
import jax.numpy as jnp
import numpy as np
from jax.experimental import pallas as pl

_DIM = 1024
_H = 16
_E = 64
_HID = 4096
_B = 2
_L = 2048
_NR = _B * _L
_U = 40
_EPS = 1e-5
_SCALE = 1.0 / math.sqrt(_E)
_QT = 256
_RT = 512
_RTM = 256


def _sample_counts():
    ws = []
    rows = jnp.arange(_L)[:, None]
    for seed in (42, 43):
        idx = jax.random.randint(jax.random.key(seed), (_L, _U), 0, _L)
        ws.append(jnp.zeros((_L, _L), jnp.float32).at[rows, idx].add(1.0))
    return jnp.stack(ws).astype(jnp.bfloat16)


def _ln(x, g, b):
    m = jnp.mean(x, axis=1, keepdims=True)
    v = jnp.mean((x - m) ** 2, axis=1, keepdims=True)
    return (x - m) / jnp.sqrt(v + _EPS) * g + b


def _ln_qkv_kernel(x_ref, g_ref, b_ref, w_ref, bias_ref, qkv_ref, xn_ref):
    xn = _ln(x_ref[0], g_ref[...], b_ref[...])
    xn_ref[0] = xn
    qkv_ref[0] = (
        jnp.dot(xn, w_ref[...], preferred_element_type=jnp.float32) + bias_ref[...]
    )


def _m_scores_kernel(q_ref, k_ref, w_ref, m_ref):
    w = w_ref[0].astype(jnp.float32)
    sampled = w > 0
    neg = jnp.full((), -jnp.inf, jnp.float32)
    for h in range(_H):
        q = q_ref[0, :, h * _E:(h + 1) * _E]
        k = k_ref[0, :, h * _E:(h + 1) * _E]
        s = jax.lax.dot_general(
            q, k, (((1,), (1,)), ((), ())), preferred_element_type=jnp.float32
        )
        mx = jnp.max(jnp.where(sampled, s, neg), axis=1)
        sm = jnp.sum(s * w, axis=1)
        m_ref[0, 0, h, :] = mx - sm * (1.0 / _L)


_HG = 8


def _attn_kernel(m_ref, q_ref, k_ref, v_ref, out_ref):
    m0 = m_ref[0, 0]
    iota_l = jax.lax.broadcasted_iota(jnp.int32, (_HG, _L), 1)
    iota_u = jax.lax.broadcasted_iota(jnp.int32, (_HG, _U), 1)

    def body(i, carry):
        m, sel = carry
        rowmax = jnp.max(m, axis=1, keepdims=True)
        first = jnp.min(
            jnp.where(m == rowmax, iota_l, _L), axis=1, keepdims=True
        )
        sel = jnp.where(iota_u == i, first, sel)
        m = jnp.where(iota_l == first, -jnp.inf, m)
        return m, sel

    _, sel = jax.lax.fori_loop(
        0, _U, body, (m0, jnp.zeros((_HG, _U), jnp.int32))
    )

    onehot_iota = jax.lax.broadcasted_iota(jnp.int32, (_U, _L), 1)
    for h in range(_HG):
        q = q_ref[0, :, h * _E:(h + 1) * _E]
        k = k_ref[0, :, h * _E:(h + 1) * _E]
        v = v_ref[0, :, h * _E:(h + 1) * _E]
        oh = (onehot_iota == sel[h][:, None]).astype(jnp.float32)
        qred = jnp.dot(oh, q, preferred_element_type=jnp.float32)
        s = jax.lax.dot_general(
            qred, k, (((1,), (1,)), ((), ())), preferred_element_type=jnp.float32
        ) * _SCALE
        s = s - jnp.max(s, axis=1, keepdims=True)
        e = jnp.exp(s)
        attn = e / jnp.sum(e, axis=1, keepdims=True)
        upd = jnp.dot(attn, v, preferred_element_type=jnp.float32)
        mean_v = jnp.mean(v, axis=0, keepdims=True)
        scat = jax.lax.dot_general(
            oh, upd, (((0,), (0,)), ((), ())), preferred_element_type=jnp.float32
        )
        ctx = scat + (1.0 - jnp.sum(oh, axis=0))[:, None] * mean_v
        out_ref[0, 0, :, h * _E:(h + 1) * _E] = ctx


def _wo_ln_kernel(ctx_ref, xn_ref, w_ref, b_ref, g_ref, bb_ref, y_ref):
    t = (
        jnp.dot(ctx_ref[0], w_ref[...], preferred_element_type=jnp.float32)
        + b_ref[...]
        + xn_ref[0]
    )
    y_ref[0] = _ln(t, g_ref[...], bb_ref[...])


def _mlp_fc1_kernel(y_ref, w1_ref, b1_ref, g1_ref, bb1_ref, h_ref):
    t = (
        jnp.dot(y_ref[0], w1_ref[...], preferred_element_type=jnp.float32)
        + b1_ref[...]
    )
    t = _ln(t, g1_ref[...], bb1_ref[...])
    h_ref[0] = t * 0.5 * (1.0 + jax.lax.erf(t * np.float32(1.0 / np.sqrt(2.0))))


def _mlp_fc2_kernel(h_ref, y_ref, w2_ref, b2_ref, g2_ref, bb2_ref, o_ref):
    t = (
        jnp.dot(h_ref[0], w2_ref[...], preferred_element_type=jnp.float32)
        + b2_ref[...]
    )
    o_ref[0] = y_ref[0] + _ln(t, g2_ref[...], bb2_ref[...])


def kernel(input, ln1_g, ln1_b, ln2_g, ln2_b, Wq, bq, Wk, bk, Wv, bv, Wo, bo,
           fc1_W, fc1_b, mln1_g, mln1_b, fc2_W, fc2_b, mln2_g, mln2_b):
    f32 = jnp.float32
    x = input.reshape(2, _NR, _DIM)
    wqkv = jnp.concatenate([Wq, Wk, Wv], axis=1)
    bqkv = jnp.concatenate([bq, bk, bv])[None, :]
    w_counts = _sample_counts()

    qkv, xn = pl.pallas_call(
        _ln_qkv_kernel,
        grid=(2, _NR // _RT),
        in_specs=[
            pl.BlockSpec((1, _RT, _DIM), lambda c, r: (c, r, 0)),
            pl.BlockSpec((1, _DIM), lambda c, r: (0, 0)),
            pl.BlockSpec((1, _DIM), lambda c, r: (0, 0)),
            pl.BlockSpec((_DIM, 3 * _DIM), lambda c, r: (0, 0)),
            pl.BlockSpec((1, 3 * _DIM), lambda c, r: (0, 0)),
        ],
        out_specs=[
            pl.BlockSpec((1, _RT, 3 * _DIM), lambda c, r: (c, r, 0)),
            pl.BlockSpec((1, _RT, _DIM), lambda c, r: (c, r, 0)),
        ],
        out_shape=[
            jax.ShapeDtypeStruct((2, _NR, 3 * _DIM), f32),
            jax.ShapeDtypeStruct((2, _NR, _DIM), f32),
        ],
    )(x, ln1_g[None, :], ln1_b[None, :], wqkv, bqkv)

    nqt = _L // _QT

    m_arr = pl.pallas_call(
        _m_scores_kernel,
        grid=(2, _B, nqt),
        in_specs=[
            pl.BlockSpec((1, _QT, _DIM),
                         lambda c, b, qt: (1 - c, b * (_L // _QT) + qt, 0)),
            pl.BlockSpec((1, _L, _DIM), lambda c, b, qt: (c, b, 1)),
            pl.BlockSpec((1, _QT, _L), lambda c, b, qt: (c, qt, 0)),
        ],
        out_specs=pl.BlockSpec((1, 1, _H, _QT), lambda c, b, qt: (c, b, 0, qt)),
        out_shape=jax.ShapeDtypeStruct((2, _B, _H, _L), f32),
    )(qkv, qkv, w_counts)

    hgw = _HG * _E
    ctx = pl.pallas_call(
        _attn_kernel,
        grid=(2, _B, _H // _HG),
        in_specs=[
            pl.BlockSpec((1, 1, _HG, _L), lambda c, b, g: (c, b, g, 0)),
            pl.BlockSpec((1, _L, hgw),
                         lambda c, b, g: (1 - c, b, g)),
            pl.BlockSpec((1, _L, hgw),
                         lambda c, b, g: (c, b, _DIM // hgw + g)),
            pl.BlockSpec((1, _L, hgw),
                         lambda c, b, g: (c, b, 2 * (_DIM // hgw) + g)),
        ],
        out_specs=pl.BlockSpec((1, 1, _L, hgw), lambda c, b, g: (c, b, 0, g)),
        out_shape=jax.ShapeDtypeStruct((2, _B, _L, _DIM), f32),
    )(m_arr, qkv, qkv, qkv)

    y = pl.pallas_call(
        _wo_ln_kernel,
        grid=(2, _NR // _RT),
        in_specs=[
            pl.BlockSpec((1, _RT, _DIM), lambda c, r: (c, r, 0)),
            pl.BlockSpec((1, _RT, _DIM), lambda c, r: (c, r, 0)),
            pl.BlockSpec((_DIM, _DIM), lambda c, r: (0, 0)),
            pl.BlockSpec((1, _DIM), lambda c, r: (0, 0)),
            pl.BlockSpec((1, _DIM), lambda c, r: (0, 0)),
            pl.BlockSpec((1, _DIM), lambda c, r: (0, 0)),
        ],
        out_specs=pl.BlockSpec((1, _RT, _DIM), lambda c, r: (c, r, 0)),
        out_shape=jax.ShapeDtypeStruct((2, _NR, _DIM), f32),
    )(ctx.reshape(2, _NR, _DIM), xn, Wo, bo[None, :], ln2_g[None, :],
      ln2_b[None, :])

    hmid = pl.pallas_call(
        _mlp_fc1_kernel,
        grid=(2, _NR // _RTM),
        in_specs=[
            pl.BlockSpec((1, _RTM, _DIM), lambda c, r: (c, r, 0)),
            pl.BlockSpec((_DIM, _HID), lambda c, r: (0, 0)),
            pl.BlockSpec((1, _HID), lambda c, r: (0, 0)),
            pl.BlockSpec((1, _HID), lambda c, r: (0, 0)),
            pl.BlockSpec((1, _HID), lambda c, r: (0, 0)),
        ],
        out_specs=pl.BlockSpec((1, _RTM, _HID), lambda c, r: (c, r, 0)),
        out_shape=jax.ShapeDtypeStruct((2, _NR, _HID), f32),
    )(y, fc1_W, fc1_b[None, :], mln1_g[None, :], mln1_b[None, :])

    out = pl.pallas_call(
        _mlp_fc2_kernel,
        grid=(2, _NR // _RTM),
        in_specs=[
            pl.BlockSpec((1, _RTM, _HID), lambda c, r: (c, r, 0)),
            pl.BlockSpec((1, _RTM, _DIM), lambda c, r: (c, r, 0)),
            pl.BlockSpec((_HID, _DIM), lambda c, r: (0, 0)),
            pl.BlockSpec((1, _DIM), lambda c, r: (0, 0)),
            pl.BlockSpec((1, _DIM), lambda c, r: (0, 0)),
            pl.BlockSpec((1, _DIM), lambda c, r: (0, 0)),
        ],
        out_specs=pl.BlockSpec((1, _RTM, _DIM), lambda c, r: (c, r, 0)),
        out_shape=jax.ShapeDtypeStruct((2, _NR, _DIM), f32),
    )(hmid, y, fc2_W, fc2_b[None, :], mln2_g[None, :], mln2_b[None, :])

    out = out.reshape(2, _B, _L, _DIM)
    return (out[0], out[1])


# speedup vs baseline: 1.0007x; 1.0007x over previous
import math

import jax
import jax.numpy as jnp
import numpy as np
from jax.experimental import pallas as pl

_DIM = 1024
_H = 16
_E = 64
_HID = 4096
_B = 2
_L = 2048
_NR = _B * _L
_U = 40
_EPS = 1e-5
_SCALE = 1.0 / math.sqrt(_E)
_QT = 256
_RT = 512
_RTM = 256


_W_CACHE = []


def _sample_counts():
    if _W_CACHE:
        return jnp.asarray(_W_CACHE[0])
    try:
        import ml_dtypes
        w = np.zeros((2, _L, _L), np.float32)
        for c, seed in enumerate((42, 43)):
            idx = np.asarray(
                jax.random.randint(jax.random.key(seed), (_L, _U), 0, _L))
            np.add.at(w[c], (np.arange(_L)[:, None], idx), 1.0)
        _W_CACHE.append(w.astype(ml_dtypes.bfloat16))
        return jnp.asarray(_W_CACHE[0])
    except Exception:
        ws = []
        rows = jnp.arange(_L)[:, None]
        for seed in (42, 43):
            idx = jax.random.randint(jax.random.key(seed), (_L, _U), 0, _L)
            ws.append(jnp.zeros((_L, _L), jnp.float32).at[rows, idx].add(1.0))
        return jnp.stack(ws).astype(jnp.bfloat16)


def _ln(x, g, b):
    m = jnp.mean(x, axis=1, keepdims=True)
    v = jnp.mean((x - m) ** 2, axis=1, keepdims=True)
    return (x - m) / jnp.sqrt(v + _EPS) * g + b


def _ln_qkv_kernel(x_ref, g_ref, b_ref, w_ref, bias_ref, qkv_ref, xn_ref):
    xn = _ln(x_ref[0], g_ref[...], b_ref[...])
    xn_ref[0] = xn
    qkv_ref[0] = (
        jnp.dot(xn, w_ref[...], preferred_element_type=jnp.float32) + bias_ref[...]
    )


def _m_scores_kernel(q_ref, k_ref, w_ref, m_ref):
    w = w_ref[0].astype(jnp.float32)
    sampled = w > 0
    neg = jnp.full((), -jnp.inf, jnp.float32)
    for h in range(_H):
        q = q_ref[0, :, h * _E:(h + 1) * _E]
        k = k_ref[0, :, h * _E:(h + 1) * _E]
        s = jax.lax.dot_general(
            q, k, (((1,), (1,)), ((), ())), preferred_element_type=jnp.float32
        )
        mx = jnp.max(jnp.where(sampled, s, neg), axis=1)
        sm = jnp.sum(s * w, axis=1)
        m_ref[0, 0, h, :] = mx - sm * (1.0 / _L)


_HG = 8


def _attn_kernel(m_ref, q_ref, k_ref, v_ref, out_ref):
    m0 = m_ref[0, 0]
    iota_l = jax.lax.broadcasted_iota(jnp.int32, (_HG, _L), 1)
    iota_u = jax.lax.broadcasted_iota(jnp.int32, (_HG, _U), 1)

    def body(i, carry):
        m, sel = carry
        rowmax = jnp.max(m, axis=1, keepdims=True)
        first = jnp.min(
            jnp.where(m == rowmax, iota_l, _L), axis=1, keepdims=True
        )
        sel = jnp.where(iota_u == i, first, sel)
        m = jnp.where(iota_l == first, -jnp.inf, m)
        return m, sel

    _, sel = jax.lax.fori_loop(
        0, _U, body, (m0, jnp.zeros((_HG, _U), jnp.int32))
    )

    onehot_iota = jax.lax.broadcasted_iota(jnp.int32, (_U, _L), 1)
    for h in range(_HG):
        q = q_ref[0, :, h * _E:(h + 1) * _E]
        k = k_ref[0, :, h * _E:(h + 1) * _E]
        v = v_ref[0, :, h * _E:(h + 1) * _E]
        oh = (onehot_iota == sel[h][:, None]).astype(jnp.float32)
        qred = jnp.dot(oh, q, preferred_element_type=jnp.float32)
        s = jax.lax.dot_general(
            qred, k, (((1,), (1,)), ((), ())), preferred_element_type=jnp.float32
        ) * _SCALE
        s = s - jnp.max(s, axis=1, keepdims=True)
        e = jnp.exp(s)
        attn = e / jnp.sum(e, axis=1, keepdims=True)
        upd = jnp.dot(attn, v, preferred_element_type=jnp.float32)
        mean_v = jnp.mean(v, axis=0, keepdims=True)
        scat = jax.lax.dot_general(
            oh, upd, (((0,), (0,)), ((), ())), preferred_element_type=jnp.float32
        )
        ctx = scat + (1.0 - jnp.sum(oh, axis=0))[:, None] * mean_v
        out_ref[0, 0, :, h * _E:(h + 1) * _E] = ctx


def _wo_ln_kernel(ctx_ref, xn_ref, w_ref, b_ref, g_ref, bb_ref, y_ref):
    t = (
        jnp.dot(ctx_ref[0], w_ref[...], preferred_element_type=jnp.float32)
        + b_ref[...]
        + xn_ref[0]
    )
    y_ref[0] = _ln(t, g_ref[...], bb_ref[...])


def _mlp_fc1_kernel(y_ref, w1_ref, b1_ref, g1_ref, bb1_ref, h_ref):
    t = (
        jnp.dot(y_ref[0], w1_ref[...], preferred_element_type=jnp.float32)
        + b1_ref[...]
    )
    t = _ln(t, g1_ref[...], bb1_ref[...])
    h_ref[0] = t * 0.5 * (1.0 + jax.lax.erf(t * np.float32(1.0 / np.sqrt(2.0))))


def _mlp_fc2_kernel(h_ref, y_ref, w2_ref, b2_ref, g2_ref, bb2_ref, o_ref):
    t = (
        jnp.dot(h_ref[0], w2_ref[...], preferred_element_type=jnp.float32)
        + b2_ref[...]
    )
    o_ref[0] = y_ref[0] + _ln(t, g2_ref[...], bb2_ref[...])


def kernel(input, ln1_g, ln1_b, ln2_g, ln2_b, Wq, bq, Wk, bk, Wv, bv, Wo, bo,
           fc1_W, fc1_b, mln1_g, mln1_b, fc2_W, fc2_b, mln2_g, mln2_b):
    f32 = jnp.float32
    x = input.reshape(2, _NR, _DIM)
    wqkv = jnp.concatenate([Wq, Wk, Wv], axis=1)
    bqkv = jnp.concatenate([bq, bk, bv])[None, :]
    w_counts = _sample_counts()

    qkv, xn = pl.pallas_call(
        _ln_qkv_kernel,
        grid=(2, _NR // _RT),
        in_specs=[
            pl.BlockSpec((1, _RT, _DIM), lambda c, r: (c, r, 0)),
            pl.BlockSpec((1, _DIM), lambda c, r: (0, 0)),
            pl.BlockSpec((1, _DIM), lambda c, r: (0, 0)),
            pl.BlockSpec((_DIM, 3 * _DIM), lambda c, r: (0, 0)),
            pl.BlockSpec((1, 3 * _DIM), lambda c, r: (0, 0)),
        ],
        out_specs=[
            pl.BlockSpec((1, _RT, 3 * _DIM), lambda c, r: (c, r, 0)),
            pl.BlockSpec((1, _RT, _DIM), lambda c, r: (c, r, 0)),
        ],
        out_shape=[
            jax.ShapeDtypeStruct((2, _NR, 3 * _DIM), f32),
            jax.ShapeDtypeStruct((2, _NR, _DIM), f32),
        ],
    )(x, ln1_g[None, :], ln1_b[None, :], wqkv, bqkv)

    nqt = _L // _QT

    m_arr = pl.pallas_call(
        _m_scores_kernel,
        grid=(2, _B, nqt),
        in_specs=[
            pl.BlockSpec((1, _QT, _DIM),
                         lambda c, b, qt: (1 - c, b * (_L // _QT) + qt, 0)),
            pl.BlockSpec((1, _L, _DIM), lambda c, b, qt: (c, b, 1)),
            pl.BlockSpec((1, _QT, _L), lambda c, b, qt: (c, qt, 0)),
        ],
        out_specs=pl.BlockSpec((1, 1, _H, _QT), lambda c, b, qt: (c, b, 0, qt)),
        out_shape=jax.ShapeDtypeStruct((2, _B, _H, _L), f32),
    )(qkv, qkv, w_counts)

    hgw = _HG * _E
    ctx = pl.pallas_call(
        _attn_kernel,
        grid=(2, _B, _H // _HG),
        in_specs=[
            pl.BlockSpec((1, 1, _HG, _L), lambda c, b, g: (c, b, g, 0)),
            pl.BlockSpec((1, _L, hgw),
                         lambda c, b, g: (1 - c, b, g)),
            pl.BlockSpec((1, _L, hgw),
                         lambda c, b, g: (c, b, _DIM // hgw + g)),
            pl.BlockSpec((1, _L, hgw),
                         lambda c, b, g: (c, b, 2 * (_DIM // hgw) + g)),
        ],
        out_specs=pl.BlockSpec((1, 1, _L, hgw), lambda c, b, g: (c, b, 0, g)),
        out_shape=jax.ShapeDtypeStruct((2, _B, _L, _DIM), f32),
    )(m_arr, qkv, qkv, qkv)

    y = pl.pallas_call(
        _wo_ln_kernel,
        grid=(2, _NR // _RT),
        in_specs=[
            pl.BlockSpec((1, _RT, _DIM), lambda c, r: (c, r, 0)),
            pl.BlockSpec((1, _RT, _DIM), lambda c, r: (c, r, 0)),
            pl.BlockSpec((_DIM, _DIM), lambda c, r: (0, 0)),
            pl.BlockSpec((1, _DIM), lambda c, r: (0, 0)),
            pl.BlockSpec((1, _DIM), lambda c, r: (0, 0)),
            pl.BlockSpec((1, _DIM), lambda c, r: (0, 0)),
        ],
        out_specs=pl.BlockSpec((1, _RT, _DIM), lambda c, r: (c, r, 0)),
        out_shape=jax.ShapeDtypeStruct((2, _NR, _DIM), f32),
    )(ctx.reshape(2, _NR, _DIM), xn, Wo, bo[None, :], ln2_g[None, :],
      ln2_b[None, :])

    hmid = pl.pallas_call(
        _mlp_fc1_kernel,
        grid=(2, _NR // _RTM),
        in_specs=[
            pl.BlockSpec((1, _RTM, _DIM), lambda c, r: (c, r, 0)),
            pl.BlockSpec((_DIM, _HID), lambda c, r: (0, 0)),
            pl.BlockSpec((1, _HID), lambda c, r: (0, 0)),
            pl.BlockSpec((1, _HID), lambda c, r: (0, 0)),
            pl.BlockSpec((1, _HID), lambda c, r: (0, 0)),
        ],
        out_specs=pl.BlockSpec((1, _RTM, _HID), lambda c, r: (c, r, 0)),
        out_shape=jax.ShapeDtypeStruct((2, _NR, _HID), f32),
    )(y, fc1_W, fc1_b[None, :], mln1_g[None, :], mln1_b[None, :])

    out = pl.pallas_call(
        _mlp_fc2_kernel,
        grid=(2, _NR // _RTM),
        in_specs=[
            pl.BlockSpec((1, _RTM, _HID), lambda c, r: (c, r, 0)),
            pl.BlockSpec((1, _RTM, _DIM), lambda c, r: (c, r, 0)),
            pl.BlockSpec((_HID, _DIM), lambda c, r: (0, 0)),
            pl.BlockSpec((1, _DIM), lambda c, r: (0, 0)),
            pl.BlockSpec((1, _DIM), lambda c, r: (0, 0)),
            pl.BlockSpec((1, _DIM), lambda c, r: (0, 0)),
        ],
        out_specs=pl.BlockSpec((1, _RTM, _DIM), lambda c, r: (c, r, 0)),
        out_shape=jax.ShapeDtypeStruct((2, _NR, _DIM), f32),
    )(hmid, y, fc2_W, fc2_b[None, :], mln2_g[None, :], mln2_b[None, :])

    out = out.reshape(2, _B, _L, _DIM)
    return (out[0], out[1])


# speedup vs baseline: 1.4380x; 1.4369x over previous
import math

import jax
import jax.numpy as jnp
import numpy as np
from jax.experimental import pallas as pl

_DIM = 1024
_H = 16
_E = 64
_HID = 4096
_B = 2
_L = 2048
_NR = _B * _L
_U = 40
_EPS = 1e-5
_SCALE = 1.0 / math.sqrt(_E)
_QT = 256
_RT = 512
_RTM = 256


_W_CACHE = []


def _sample_counts():
    if _W_CACHE:
        return jnp.asarray(_W_CACHE[0])
    try:
        import ml_dtypes
        w = np.zeros((2, _L, _L), np.float32)
        for c, seed in enumerate((42, 43)):
            with jax.ensure_compile_time_eval():
                idx = np.asarray(
                    jax.random.randint(jax.random.key(seed), (_L, _U), 0, _L))
            np.add.at(w[c], (np.arange(_L)[:, None], idx), 1.0)
        _W_CACHE.append(w.astype(ml_dtypes.bfloat16))
        return jnp.asarray(_W_CACHE[0])
    except Exception:
        ws = []
        rows = jnp.arange(_L)[:, None]
        for seed in (42, 43):
            idx = jax.random.randint(jax.random.key(seed), (_L, _U), 0, _L)
            ws.append(jnp.zeros((_L, _L), jnp.float32).at[rows, idx].add(1.0))
        return jnp.stack(ws).astype(jnp.bfloat16)


def _ln(x, g, b):
    m = jnp.mean(x, axis=1, keepdims=True)
    v = jnp.mean((x - m) ** 2, axis=1, keepdims=True)
    return (x - m) / jnp.sqrt(v + _EPS) * g + b


def _ln_qkv_kernel(x_ref, g_ref, b_ref, w_ref, bias_ref, qkv_ref, xn_ref):
    xn = _ln(x_ref[0], g_ref[...], b_ref[...])
    xn_ref[0] = xn
    qkv_ref[0] = (
        jnp.dot(xn, w_ref[...], preferred_element_type=jnp.float32) + bias_ref[...]
    )


def _m_scores_kernel(q_ref, k_ref, w_ref, m_ref):
    w = w_ref[0].astype(jnp.float32)
    sampled = w > 0
    neg = jnp.full((), -jnp.inf, jnp.float32)
    for h in range(_H):
        q = q_ref[0, :, h * _E:(h + 1) * _E]
        k = k_ref[0, :, h * _E:(h + 1) * _E]
        s = jax.lax.dot_general(
            q, k, (((1,), (1,)), ((), ())), preferred_element_type=jnp.float32
        )
        mx = jnp.max(jnp.where(sampled, s, neg), axis=1)
        sm = jnp.sum(s * w, axis=1)
        m_ref[0, 0, h, :] = mx - sm * (1.0 / _L)


_HG = 8


def _attn_kernel(m_ref, q_ref, k_ref, v_ref, out_ref):
    m0 = m_ref[0, 0]
    iota_l = jax.lax.broadcasted_iota(jnp.int32, (_HG, _L), 1)
    iota_u = jax.lax.broadcasted_iota(jnp.int32, (_HG, _U), 1)

    def body(i, carry):
        m, sel = carry
        rowmax = jnp.max(m, axis=1, keepdims=True)
        first = jnp.min(
            jnp.where(m == rowmax, iota_l, _L), axis=1, keepdims=True
        )
        sel = jnp.where(iota_u == i, first, sel)
        m = jnp.where(iota_l == first, -jnp.inf, m)
        return m, sel

    _, sel = jax.lax.fori_loop(
        0, _U, body, (m0, jnp.zeros((_HG, _U), jnp.int32))
    )

    onehot_iota = jax.lax.broadcasted_iota(jnp.int32, (_U, _L), 1)
    for h in range(_HG):
        q = q_ref[0, :, h * _E:(h + 1) * _E]
        k = k_ref[0, :, h * _E:(h + 1) * _E]
        v = v_ref[0, :, h * _E:(h + 1) * _E]
        oh = (onehot_iota == sel[h][:, None]).astype(jnp.float32)
        qred = jnp.dot(oh, q, preferred_element_type=jnp.float32)
        s = jax.lax.dot_general(
            qred, k, (((1,), (1,)), ((), ())), preferred_element_type=jnp.float32
        ) * _SCALE
        s = s - jnp.max(s, axis=1, keepdims=True)
        e = jnp.exp(s)
        attn = e / jnp.sum(e, axis=1, keepdims=True)
        upd = jnp.dot(attn, v, preferred_element_type=jnp.float32)
        mean_v = jnp.mean(v, axis=0, keepdims=True)
        scat = jax.lax.dot_general(
            oh, upd, (((0,), (0,)), ((), ())), preferred_element_type=jnp.float32
        )
        ctx = scat + (1.0 - jnp.sum(oh, axis=0))[:, None] * mean_v
        out_ref[0, 0, :, h * _E:(h + 1) * _E] = ctx


def _wo_ln_kernel(ctx_ref, xn_ref, w_ref, b_ref, g_ref, bb_ref, y_ref):
    t = (
        jnp.dot(ctx_ref[0], w_ref[...], preferred_element_type=jnp.float32)
        + b_ref[...]
        + xn_ref[0]
    )
    y_ref[0] = _ln(t, g_ref[...], bb_ref[...])


def _mlp_fc1_kernel(y_ref, w1_ref, b1_ref, g1_ref, bb1_ref, h_ref):
    t = (
        jnp.dot(y_ref[0], w1_ref[...], preferred_element_type=jnp.float32)
        + b1_ref[...]
    )
    t = _ln(t, g1_ref[...], bb1_ref[...])
    h_ref[0] = t * 0.5 * (1.0 + jax.lax.erf(t * np.float32(1.0 / np.sqrt(2.0))))


def _mlp_fc2_kernel(h_ref, y_ref, w2_ref, b2_ref, g2_ref, bb2_ref, o_ref):
    t = (
        jnp.dot(h_ref[0], w2_ref[...], preferred_element_type=jnp.float32)
        + b2_ref[...]
    )
    o_ref[0] = y_ref[0] + _ln(t, g2_ref[...], bb2_ref[...])


def kernel(input, ln1_g, ln1_b, ln2_g, ln2_b, Wq, bq, Wk, bk, Wv, bv, Wo, bo,
           fc1_W, fc1_b, mln1_g, mln1_b, fc2_W, fc2_b, mln2_g, mln2_b):
    f32 = jnp.float32
    x = input.reshape(2, _NR, _DIM)
    wqkv = jnp.concatenate([Wq, Wk, Wv], axis=1)
    bqkv = jnp.concatenate([bq, bk, bv])[None, :]
    w_counts = _sample_counts()

    qkv, xn = pl.pallas_call(
        _ln_qkv_kernel,
        grid=(2, _NR // _RT),
        in_specs=[
            pl.BlockSpec((1, _RT, _DIM), lambda c, r: (c, r, 0)),
            pl.BlockSpec((1, _DIM), lambda c, r: (0, 0)),
            pl.BlockSpec((1, _DIM), lambda c, r: (0, 0)),
            pl.BlockSpec((_DIM, 3 * _DIM), lambda c, r: (0, 0)),
            pl.BlockSpec((1, 3 * _DIM), lambda c, r: (0, 0)),
        ],
        out_specs=[
            pl.BlockSpec((1, _RT, 3 * _DIM), lambda c, r: (c, r, 0)),
            pl.BlockSpec((1, _RT, _DIM), lambda c, r: (c, r, 0)),
        ],
        out_shape=[
            jax.ShapeDtypeStruct((2, _NR, 3 * _DIM), f32),
            jax.ShapeDtypeStruct((2, _NR, _DIM), f32),
        ],
    )(x, ln1_g[None, :], ln1_b[None, :], wqkv, bqkv)

    nqt = _L // _QT

    m_arr = pl.pallas_call(
        _m_scores_kernel,
        grid=(2, _B, nqt),
        in_specs=[
            pl.BlockSpec((1, _QT, _DIM),
                         lambda c, b, qt: (1 - c, b * (_L // _QT) + qt, 0)),
            pl.BlockSpec((1, _L, _DIM), lambda c, b, qt: (c, b, 1)),
            pl.BlockSpec((1, _QT, _L), lambda c, b, qt: (c, qt, 0)),
        ],
        out_specs=pl.BlockSpec((1, 1, _H, _QT), lambda c, b, qt: (c, b, 0, qt)),
        out_shape=jax.ShapeDtypeStruct((2, _B, _H, _L), f32),
    )(qkv, qkv, w_counts)

    hgw = _HG * _E
    ctx = pl.pallas_call(
        _attn_kernel,
        grid=(2, _B, _H // _HG),
        in_specs=[
            pl.BlockSpec((1, 1, _HG, _L), lambda c, b, g: (c, b, g, 0)),
            pl.BlockSpec((1, _L, hgw),
                         lambda c, b, g: (1 - c, b, g)),
            pl.BlockSpec((1, _L, hgw),
                         lambda c, b, g: (c, b, _DIM // hgw + g)),
            pl.BlockSpec((1, _L, hgw),
                         lambda c, b, g: (c, b, 2 * (_DIM // hgw) + g)),
        ],
        out_specs=pl.BlockSpec((1, 1, _L, hgw), lambda c, b, g: (c, b, 0, g)),
        out_shape=jax.ShapeDtypeStruct((2, _B, _L, _DIM), f32),
    )(m_arr, qkv, qkv, qkv)

    y = pl.pallas_call(
        _wo_ln_kernel,
        grid=(2, _NR // _RT),
        in_specs=[
            pl.BlockSpec((1, _RT, _DIM), lambda c, r: (c, r, 0)),
            pl.BlockSpec((1, _RT, _DIM), lambda c, r: (c, r, 0)),
            pl.BlockSpec((_DIM, _DIM), lambda c, r: (0, 0)),
            pl.BlockSpec((1, _DIM), lambda c, r: (0, 0)),
            pl.BlockSpec((1, _DIM), lambda c, r: (0, 0)),
            pl.BlockSpec((1, _DIM), lambda c, r: (0, 0)),
        ],
        out_specs=pl.BlockSpec((1, _RT, _DIM), lambda c, r: (c, r, 0)),
        out_shape=jax.ShapeDtypeStruct((2, _NR, _DIM), f32),
    )(ctx.reshape(2, _NR, _DIM), xn, Wo, bo[None, :], ln2_g[None, :],
      ln2_b[None, :])

    hmid = pl.pallas_call(
        _mlp_fc1_kernel,
        grid=(2, _NR // _RTM),
        in_specs=[
            pl.BlockSpec((1, _RTM, _DIM), lambda c, r: (c, r, 0)),
            pl.BlockSpec((_DIM, _HID), lambda c, r: (0, 0)),
            pl.BlockSpec((1, _HID), lambda c, r: (0, 0)),
            pl.BlockSpec((1, _HID), lambda c, r: (0, 0)),
            pl.BlockSpec((1, _HID), lambda c, r: (0, 0)),
        ],
        out_specs=pl.BlockSpec((1, _RTM, _HID), lambda c, r: (c, r, 0)),
        out_shape=jax.ShapeDtypeStruct((2, _NR, _HID), f32),
    )(y, fc1_W, fc1_b[None, :], mln1_g[None, :], mln1_b[None, :])

    out = pl.pallas_call(
        _mlp_fc2_kernel,
        grid=(2, _NR // _RTM),
        in_specs=[
            pl.BlockSpec((1, _RTM, _HID), lambda c, r: (c, r, 0)),
            pl.BlockSpec((1, _RTM, _DIM), lambda c, r: (c, r, 0)),
            pl.BlockSpec((_HID, _DIM), lambda c, r: (0, 0)),
            pl.BlockSpec((1, _DIM), lambda c, r: (0, 0)),
            pl.BlockSpec((1, _DIM), lambda c, r: (0, 0)),
            pl.BlockSpec((1, _DIM), lambda c, r: (0, 0)),
        ],
        out_specs=pl.BlockSpec((1, _RTM, _DIM), lambda c, r: (c, r, 0)),
        out_shape=jax.ShapeDtypeStruct((2, _NR, _DIM), f32),
    )(hmid, y, fc2_W, fc2_b[None, :], mln2_g[None, :], mln2_b[None, :])

    out = out.reshape(2, _B, _L, _DIM)
    return (out[0], out[1])


# speedup vs baseline: 1.5402x; 1.0711x over previous
import math

import jax
import jax.numpy as jnp
import numpy as np
from jax.experimental import pallas as pl

_DIM = 1024
_H = 16
_E = 64
_HID = 4096
_B = 2
_L = 2048
_NR = _B * _L
_U = 40
_EPS = 1e-5
_SCALE = 1.0 / math.sqrt(_E)
_QT = 256
_RT = 512
_RTM = 256


_W_CACHE = []


def _sample_counts():
    if _W_CACHE:
        return jnp.asarray(_W_CACHE[0])
    try:
        import ml_dtypes
        w = np.zeros((2, _L, _L), np.float32)
        for c, seed in enumerate((42, 43)):
            with jax.ensure_compile_time_eval():
                idx = np.asarray(
                    jax.random.randint(jax.random.key(seed), (_L, _U), 0, _L))
            np.add.at(w[c], (np.arange(_L)[:, None], idx), 1.0)
        _W_CACHE.append(w.astype(ml_dtypes.bfloat16))
        return jnp.asarray(_W_CACHE[0])
    except Exception:
        ws = []
        rows = jnp.arange(_L)[:, None]
        for seed in (42, 43):
            idx = jax.random.randint(jax.random.key(seed), (_L, _U), 0, _L)
            ws.append(jnp.zeros((_L, _L), jnp.float32).at[rows, idx].add(1.0))
        return jnp.stack(ws).astype(jnp.bfloat16)


def _ln(x, g, b):
    m = jnp.mean(x, axis=1, keepdims=True)
    v = jnp.mean((x - m) ** 2, axis=1, keepdims=True)
    return (x - m) / jnp.sqrt(v + _EPS) * g + b


def _ln_qkv_kernel(x_ref, g_ref, b_ref, w_ref, bias_ref, qkv_ref, xn_ref):
    xn = _ln(x_ref[0], g_ref[...], b_ref[...])
    xn_ref[0] = xn
    qkv_ref[0] = (
        jnp.dot(xn, w_ref[...], preferred_element_type=jnp.float32) + bias_ref[...]
    )


def _m_scores_kernel(q_ref, k_ref, w_ref, m_ref):
    w = w_ref[0].astype(jnp.float32)
    sampled = w > 0
    neg = jnp.full((), -jnp.inf, jnp.float32)
    for h in range(_H):
        q = q_ref[0, :, h * _E:(h + 1) * _E]
        k = k_ref[0, :, h * _E:(h + 1) * _E]
        s = jax.lax.dot_general(
            q, k, (((1,), (1,)), ((), ())), preferred_element_type=jnp.float32
        )
        mx = jnp.max(jnp.where(sampled, s, neg), axis=1)
        sm = jnp.sum(s * w, axis=1)
        m_ref[0, 0, h, :] = mx - sm * (1.0 / _L)


_HG = 8


def _attn_kernel(m_ref, q_ref, k_ref, v_ref, out_ref):
    m0 = m_ref[0, 0]
    iota_l = jax.lax.broadcasted_iota(jnp.int32, (_HG, _L), 1)
    iota_u = jax.lax.broadcasted_iota(jnp.int32, (_HG, _U), 1)

    def body(i, carry):
        m, sel = carry
        rowmax = jnp.max(m, axis=1, keepdims=True)
        first = jnp.min(
            jnp.where(m == rowmax, iota_l, _L), axis=1, keepdims=True
        )
        sel = jnp.where(iota_u == i, first, sel)
        m = jnp.where(iota_l == first, -jnp.inf, m)
        return m, sel

    _, sel = jax.lax.fori_loop(
        0, _U, body, (m0, jnp.zeros((_HG, _U), jnp.int32))
    )

    onehot_iota = jax.lax.broadcasted_iota(jnp.int32, (_U, _L), 1)
    for h in range(_HG):
        q = q_ref[0, :, h * _E:(h + 1) * _E]
        k = k_ref[0, :, h * _E:(h + 1) * _E]
        v = v_ref[0, :, h * _E:(h + 1) * _E]
        oh = (onehot_iota == sel[h][:, None]).astype(jnp.float32)
        qred = jnp.dot(oh, q, preferred_element_type=jnp.float32)
        s = jax.lax.dot_general(
            qred, k, (((1,), (1,)), ((), ())), preferred_element_type=jnp.float32
        ) * _SCALE
        s = s - jnp.max(s, axis=1, keepdims=True)
        e = jnp.exp(s)
        attn = e / jnp.sum(e, axis=1, keepdims=True)
        upd = jnp.dot(attn, v, preferred_element_type=jnp.float32)
        mean_v = jnp.mean(v, axis=0, keepdims=True)
        scat = jax.lax.dot_general(
            oh, upd, (((0,), (0,)), ((), ())), preferred_element_type=jnp.float32
        )
        ctx = scat + (1.0 - jnp.sum(oh, axis=0))[:, None] * mean_v
        out_ref[0, 0, :, h * _E:(h + 1) * _E] = ctx


def _wo_ln_kernel(ctx_ref, xn_ref, w_ref, b_ref, g_ref, bb_ref, y_ref):
    t = (
        jnp.dot(ctx_ref[0].astype(jnp.bfloat16), w_ref[...],
                preferred_element_type=jnp.float32)
        + b_ref[...]
        + xn_ref[0]
    )
    y_ref[0] = _ln(t, g_ref[...], bb_ref[...])


def _mlp_kernel(y_ref, w1_ref, b1_ref, g1_ref, bb1_ref, w2_ref, b2_ref,
                g2_ref, bb2_ref, o_ref):
    y = y_ref[0]
    t = (
        jnp.dot(y.astype(jnp.bfloat16), w1_ref[...],
                preferred_element_type=jnp.float32)
        + b1_ref[...]
    )
    t = _ln(t, g1_ref[...], bb1_ref[...])
    t = t * 0.5 * (1.0 + jax.lax.erf(t * np.float32(1.0 / np.sqrt(2.0))))
    t = (
        jnp.dot(t.astype(jnp.bfloat16), w2_ref[...],
                preferred_element_type=jnp.float32)
        + b2_ref[...]
    )
    o_ref[0] = y + _ln(t, g2_ref[...], bb2_ref[...])


def kernel(input, ln1_g, ln1_b, ln2_g, ln2_b, Wq, bq, Wk, bk, Wv, bv, Wo, bo,
           fc1_W, fc1_b, mln1_g, mln1_b, fc2_W, fc2_b, mln2_g, mln2_b):
    f32 = jnp.float32
    x = input.reshape(2, _NR, _DIM)
    wqkv = jnp.concatenate([Wq, Wk, Wv], axis=1)
    bqkv = jnp.concatenate([bq, bk, bv])[None, :]
    w_counts = _sample_counts()

    qkv, xn = pl.pallas_call(
        _ln_qkv_kernel,
        grid=(2, _NR // _RT),
        in_specs=[
            pl.BlockSpec((1, _RT, _DIM), lambda c, r: (c, r, 0)),
            pl.BlockSpec((1, _DIM), lambda c, r: (0, 0)),
            pl.BlockSpec((1, _DIM), lambda c, r: (0, 0)),
            pl.BlockSpec((_DIM, 3 * _DIM), lambda c, r: (0, 0)),
            pl.BlockSpec((1, 3 * _DIM), lambda c, r: (0, 0)),
        ],
        out_specs=[
            pl.BlockSpec((1, _RT, 3 * _DIM), lambda c, r: (c, r, 0)),
            pl.BlockSpec((1, _RT, _DIM), lambda c, r: (c, r, 0)),
        ],
        out_shape=[
            jax.ShapeDtypeStruct((2, _NR, 3 * _DIM), f32),
            jax.ShapeDtypeStruct((2, _NR, _DIM), f32),
        ],
    )(x, ln1_g[None, :], ln1_b[None, :], wqkv, bqkv)

    nqt = _L // _QT

    m_arr = pl.pallas_call(
        _m_scores_kernel,
        grid=(2, _B, nqt),
        in_specs=[
            pl.BlockSpec((1, _QT, _DIM),
                         lambda c, b, qt: (1 - c, b * (_L // _QT) + qt, 0)),
            pl.BlockSpec((1, _L, _DIM), lambda c, b, qt: (c, b, 1)),
            pl.BlockSpec((1, _QT, _L), lambda c, b, qt: (c, qt, 0)),
        ],
        out_specs=pl.BlockSpec((1, 1, _H, _QT), lambda c, b, qt: (c, b, 0, qt)),
        out_shape=jax.ShapeDtypeStruct((2, _B, _H, _L), f32),
    )(qkv, qkv, w_counts)

    hgw = _HG * _E
    ctx = pl.pallas_call(
        _attn_kernel,
        grid=(2, _B, _H // _HG),
        in_specs=[
            pl.BlockSpec((1, 1, _HG, _L), lambda c, b, g: (c, b, g, 0)),
            pl.BlockSpec((1, _L, hgw),
                         lambda c, b, g: (1 - c, b, g)),
            pl.BlockSpec((1, _L, hgw),
                         lambda c, b, g: (c, b, _DIM // hgw + g)),
            pl.BlockSpec((1, _L, hgw),
                         lambda c, b, g: (c, b, 2 * (_DIM // hgw) + g)),
        ],
        out_specs=pl.BlockSpec((1, 1, _L, hgw), lambda c, b, g: (c, b, 0, g)),
        out_shape=jax.ShapeDtypeStruct((2, _B, _L, _DIM), f32),
    )(m_arr, qkv, qkv, qkv)

    y = pl.pallas_call(
        _wo_ln_kernel,
        grid=(2, _NR // _RT),
        in_specs=[
            pl.BlockSpec((1, _RT, _DIM), lambda c, r: (c, r, 0)),
            pl.BlockSpec((1, _RT, _DIM), lambda c, r: (c, r, 0)),
            pl.BlockSpec((_DIM, _DIM), lambda c, r: (0, 0)),
            pl.BlockSpec((1, _DIM), lambda c, r: (0, 0)),
            pl.BlockSpec((1, _DIM), lambda c, r: (0, 0)),
            pl.BlockSpec((1, _DIM), lambda c, r: (0, 0)),
        ],
        out_specs=pl.BlockSpec((1, _RT, _DIM), lambda c, r: (c, r, 0)),
        out_shape=jax.ShapeDtypeStruct((2, _NR, _DIM), f32),
    )(ctx.reshape(2, _NR, _DIM), xn, Wo.astype(jnp.bfloat16), bo[None, :],
      ln2_g[None, :], ln2_b[None, :])

    out = pl.pallas_call(
        _mlp_kernel,
        grid=(2, _NR // _RTM),
        in_specs=[
            pl.BlockSpec((1, _RTM, _DIM), lambda c, r: (c, r, 0)),
            pl.BlockSpec((_DIM, _HID), lambda c, r: (0, 0)),
            pl.BlockSpec((1, _HID), lambda c, r: (0, 0)),
            pl.BlockSpec((1, _HID), lambda c, r: (0, 0)),
            pl.BlockSpec((1, _HID), lambda c, r: (0, 0)),
            pl.BlockSpec((_HID, _DIM), lambda c, r: (0, 0)),
            pl.BlockSpec((1, _DIM), lambda c, r: (0, 0)),
            pl.BlockSpec((1, _DIM), lambda c, r: (0, 0)),
            pl.BlockSpec((1, _DIM), lambda c, r: (0, 0)),
        ],
        out_specs=pl.BlockSpec((1, _RTM, _DIM), lambda c, r: (c, r, 0)),
        out_shape=jax.ShapeDtypeStruct((2, _NR, _DIM), f32),
    )(y, fc1_W.astype(jnp.bfloat16), fc1_b[None, :], mln1_g[None, :],
      mln1_b[None, :], fc2_W.astype(jnp.bfloat16), fc2_b[None, :],
      mln2_g[None, :], mln2_b[None, :])

    out = out.reshape(2, _B, _L, _DIM)
    return (out[0], out[1])


# speedup vs baseline: 1.6714x; 1.0852x over previous
import math

import jax
import jax.numpy as jnp
import numpy as np
from jax.experimental import pallas as pl

_DIM = 1024
_H = 16
_E = 64
_HID = 4096
_B = 2
_L = 2048
_NR = _B * _L
_U = 40
_EPS = 1e-5
_SCALE = 1.0 / math.sqrt(_E)
_QT = 256
_RT = 512
_RTM = 256


_W_CACHE = []


def _sample_counts():
    if _W_CACHE:
        return jnp.asarray(_W_CACHE[0])
    try:
        import ml_dtypes
        w = np.zeros((2, _L, _L), np.float32)
        for c, seed in enumerate((42, 43)):
            with jax.ensure_compile_time_eval():
                idx = np.asarray(
                    jax.random.randint(jax.random.key(seed), (_L, _U), 0, _L))
            np.add.at(w[c], (np.arange(_L)[:, None], idx), 1.0)
        _W_CACHE.append(w.astype(ml_dtypes.bfloat16))
        return jnp.asarray(_W_CACHE[0])
    except Exception:
        ws = []
        rows = jnp.arange(_L)[:, None]
        for seed in (42, 43):
            idx = jax.random.randint(jax.random.key(seed), (_L, _U), 0, _L)
            ws.append(jnp.zeros((_L, _L), jnp.float32).at[rows, idx].add(1.0))
        return jnp.stack(ws).astype(jnp.bfloat16)


def _ln(x, g, b):
    m = jnp.mean(x, axis=1, keepdims=True)
    v = jnp.mean((x - m) ** 2, axis=1, keepdims=True)
    return (x - m) / jnp.sqrt(v + _EPS) * g + b


def _ln_qkv_kernel(x_ref, g_ref, b_ref, w_ref, bias_ref, qkv_ref, xn_ref):
    xn = _ln(x_ref[0], g_ref[...], b_ref[...])
    xn_ref[0] = xn
    qkv_ref[0] = (
        jnp.dot(xn, w_ref[...], preferred_element_type=jnp.float32) + bias_ref[...]
    )


def _m_scores_kernel(q_ref, k_ref, w_ref, m_ref):
    w = w_ref[0].astype(jnp.float32)
    sampled = w > 0
    neg = jnp.full((), -jnp.inf, jnp.float32)
    for h in range(_H):
        q = q_ref[0, :, h * _E:(h + 1) * _E]
        k = k_ref[0, :, h * _E:(h + 1) * _E]
        s = jax.lax.dot_general(
            q, k, (((1,), (1,)), ((), ())), preferred_element_type=jnp.float32
        )
        mx = jnp.max(jnp.where(sampled, s, neg), axis=1)
        sm = jnp.sum(s * w, axis=1)
        m_ref[0, 0, h, :] = mx - sm * (1.0 / _L)


_HG = 8
_NBH = 2 * _B * _H


def _topk_kernel(m_ref, sel_ref):
    m = m_ref[...].reshape(_NBH, _L)
    iota_l = jax.lax.broadcasted_iota(jnp.int32, (_NBH, _L), 1)
    iota_u = jax.lax.broadcasted_iota(jnp.int32, (_NBH, _U), 1)

    def body(i, carry):
        m, sel = carry
        rowmax = jnp.max(m, axis=1, keepdims=True)
        first = jnp.min(
            jnp.where(m == rowmax, iota_l, _L), axis=1, keepdims=True
        )
        sel = jnp.where(iota_u == i, first, sel)
        m = jnp.where(iota_l == first, -jnp.inf, m)
        return m, sel

    _, sel = jax.lax.fori_loop(
        0, _U, body, (m, jnp.zeros((_NBH, _U), jnp.int32))
    )
    sel_ref[...] = sel.reshape(2, _B, _H, _U)


def _attn_kernel(sel_ref, q_ref, k_ref, v_ref, out_ref):
    sel = sel_ref[0, 0]
    onehot_iota = jax.lax.broadcasted_iota(jnp.int32, (_U, _L), 1)
    for h in range(_HG):
        q = q_ref[0, :, h * _E:(h + 1) * _E]
        k = k_ref[0, :, h * _E:(h + 1) * _E]
        v = v_ref[0, :, h * _E:(h + 1) * _E]
        oh = (onehot_iota == sel[h][:, None]).astype(jnp.float32)
        qred = jnp.dot(oh, q, preferred_element_type=jnp.float32)
        s = jax.lax.dot_general(
            qred, k, (((1,), (1,)), ((), ())), preferred_element_type=jnp.float32
        ) * _SCALE
        s = s - jnp.max(s, axis=1, keepdims=True)
        e = jnp.exp(s)
        attn = e / jnp.sum(e, axis=1, keepdims=True)
        upd = jnp.dot(attn, v, preferred_element_type=jnp.float32)
        mean_v = jnp.mean(v, axis=0, keepdims=True)
        scat = jax.lax.dot_general(
            oh, upd, (((0,), (0,)), ((), ())), preferred_element_type=jnp.float32
        )
        ctx = scat + (1.0 - jnp.sum(oh, axis=0))[:, None] * mean_v
        out_ref[0, 0, :, h * _E:(h + 1) * _E] = ctx.astype(jnp.bfloat16)


def _wo_ln_kernel(ctx_ref, xn_ref, w_ref, b_ref, g_ref, bb_ref, y_ref):
    t = (
        jnp.dot(ctx_ref[0], w_ref[...],
                preferred_element_type=jnp.float32)
        + b_ref[...]
        + xn_ref[0]
    )
    y_ref[0] = _ln(t, g_ref[...], bb_ref[...])


def _mlp_kernel(y_ref, w1_ref, b1_ref, g1_ref, bb1_ref, w2_ref, b2_ref,
                g2_ref, bb2_ref, o_ref):
    y = y_ref[0]
    t = (
        jnp.dot(y.astype(jnp.bfloat16), w1_ref[...],
                preferred_element_type=jnp.float32)
        + b1_ref[...]
    )
    t = _ln(t, g1_ref[...], bb1_ref[...])
    t = t * 0.5 * (1.0 + jax.lax.erf(t * np.float32(1.0 / np.sqrt(2.0))))
    t = (
        jnp.dot(t.astype(jnp.bfloat16), w2_ref[...],
                preferred_element_type=jnp.float32)
        + b2_ref[...]
    )
    o_ref[0] = y + _ln(t, g2_ref[...], bb2_ref[...])


def kernel(input, ln1_g, ln1_b, ln2_g, ln2_b, Wq, bq, Wk, bk, Wv, bv, Wo, bo,
           fc1_W, fc1_b, mln1_g, mln1_b, fc2_W, fc2_b, mln2_g, mln2_b):
    f32 = jnp.float32
    x = input.reshape(2, _NR, _DIM)
    wqkv = jnp.concatenate([Wq, Wk, Wv], axis=1)
    bqkv = jnp.concatenate([bq, bk, bv])[None, :]
    w_counts = _sample_counts()

    qkv, xn = pl.pallas_call(
        _ln_qkv_kernel,
        grid=(2, _NR // _RT),
        in_specs=[
            pl.BlockSpec((1, _RT, _DIM), lambda c, r: (c, r, 0)),
            pl.BlockSpec((1, _DIM), lambda c, r: (0, 0)),
            pl.BlockSpec((1, _DIM), lambda c, r: (0, 0)),
            pl.BlockSpec((_DIM, 3 * _DIM), lambda c, r: (0, 0)),
            pl.BlockSpec((1, 3 * _DIM), lambda c, r: (0, 0)),
        ],
        out_specs=[
            pl.BlockSpec((1, _RT, 3 * _DIM), lambda c, r: (c, r, 0)),
            pl.BlockSpec((1, _RT, _DIM), lambda c, r: (c, r, 0)),
        ],
        out_shape=[
            jax.ShapeDtypeStruct((2, _NR, 3 * _DIM), f32),
            jax.ShapeDtypeStruct((2, _NR, _DIM), f32),
        ],
    )(x, ln1_g[None, :], ln1_b[None, :], wqkv, bqkv)

    nqt = _L // _QT

    m_arr = pl.pallas_call(
        _m_scores_kernel,
        grid=(2, _B, nqt),
        in_specs=[
            pl.BlockSpec((1, _QT, _DIM),
                         lambda c, b, qt: (1 - c, b * (_L // _QT) + qt, 0)),
            pl.BlockSpec((1, _L, _DIM), lambda c, b, qt: (c, b, 1)),
            pl.BlockSpec((1, _QT, _L), lambda c, b, qt: (c, qt, 0)),
        ],
        out_specs=pl.BlockSpec((1, 1, _H, _QT), lambda c, b, qt: (c, b, 0, qt)),
        out_shape=jax.ShapeDtypeStruct((2, _B, _H, _L), f32),
    )(qkv, qkv, w_counts)

    sel = pl.pallas_call(
        _topk_kernel,
        grid=(1,),
        in_specs=[pl.BlockSpec((2, _B, _H, _L), lambda i: (0, 0, 0, 0))],
        out_specs=pl.BlockSpec((2, _B, _H, _U), lambda i: (0, 0, 0, 0)),
        out_shape=jax.ShapeDtypeStruct((2, _B, _H, _U), jnp.int32),
    )(m_arr)

    hgw = _HG * _E
    ctx = pl.pallas_call(
        _attn_kernel,
        grid=(2, _B, _H // _HG),
        in_specs=[
            pl.BlockSpec((1, 1, _HG, _U), lambda c, b, g: (c, b, g, 0)),
            pl.BlockSpec((1, _L, hgw),
                         lambda c, b, g: (1 - c, b, g)),
            pl.BlockSpec((1, _L, hgw),
                         lambda c, b, g: (c, b, _DIM // hgw + g)),
            pl.BlockSpec((1, _L, hgw),
                         lambda c, b, g: (c, b, 2 * (_DIM // hgw) + g)),
        ],
        out_specs=pl.BlockSpec((1, 1, _L, hgw), lambda c, b, g: (c, b, 0, g)),
        out_shape=jax.ShapeDtypeStruct((2, _B, _L, _DIM), jnp.bfloat16),
    )(sel, qkv, qkv, qkv)

    y = pl.pallas_call(
        _wo_ln_kernel,
        grid=(2, _NR // _RT),
        in_specs=[
            pl.BlockSpec((1, _RT, _DIM), lambda c, r: (c, r, 0)),
            pl.BlockSpec((1, _RT, _DIM), lambda c, r: (c, r, 0)),
            pl.BlockSpec((_DIM, _DIM), lambda c, r: (0, 0)),
            pl.BlockSpec((1, _DIM), lambda c, r: (0, 0)),
            pl.BlockSpec((1, _DIM), lambda c, r: (0, 0)),
            pl.BlockSpec((1, _DIM), lambda c, r: (0, 0)),
        ],
        out_specs=pl.BlockSpec((1, _RT, _DIM), lambda c, r: (c, r, 0)),
        out_shape=jax.ShapeDtypeStruct((2, _NR, _DIM), f32),
    )(ctx.reshape(2, _NR, _DIM), xn, Wo.astype(jnp.bfloat16), bo[None, :],
      ln2_g[None, :], ln2_b[None, :])

    out = pl.pallas_call(
        _mlp_kernel,
        grid=(2, _NR // _RTM),
        in_specs=[
            pl.BlockSpec((1, _RTM, _DIM), lambda c, r: (c, r, 0)),
            pl.BlockSpec((_DIM, _HID), lambda c, r: (0, 0)),
            pl.BlockSpec((1, _HID), lambda c, r: (0, 0)),
            pl.BlockSpec((1, _HID), lambda c, r: (0, 0)),
            pl.BlockSpec((1, _HID), lambda c, r: (0, 0)),
            pl.BlockSpec((_HID, _DIM), lambda c, r: (0, 0)),
            pl.BlockSpec((1, _DIM), lambda c, r: (0, 0)),
            pl.BlockSpec((1, _DIM), lambda c, r: (0, 0)),
            pl.BlockSpec((1, _DIM), lambda c, r: (0, 0)),
        ],
        out_specs=pl.BlockSpec((1, _RTM, _DIM), lambda c, r: (c, r, 0)),
        out_shape=jax.ShapeDtypeStruct((2, _NR, _DIM), f32),
    )(y, fc1_W.astype(jnp.bfloat16), fc1_b[None, :], mln1_g[None, :],
      mln1_b[None, :], fc2_W.astype(jnp.bfloat16), fc2_b[None, :],
      mln2_g[None, :], mln2_b[None, :])

    out = out.reshape(2, _B, _L, _DIM)
    return (out[0], out[1])


# speedup vs baseline: 1.6807x; 1.0056x over previous
import math

import jax
import jax.numpy as jnp
import numpy as np
from jax.experimental import pallas as pl

_DIM = 1024
_H = 16
_E = 64
_HID = 4096
_B = 2
_L = 2048
_NR = _B * _L
_U = 40
_EPS = 1e-5
_SCALE = 1.0 / math.sqrt(_E)
_QT = 256
_RT = 512
_RTM = 256


_W_CACHE = []


def _sample_counts():
    if _W_CACHE:
        return jnp.asarray(_W_CACHE[0]), jnp.asarray(_W_CACHE[1])
    try:
        import ml_dtypes
        w = np.zeros((2, _L, _L), np.float32)
        for c, seed in enumerate((42, 43)):
            with jax.ensure_compile_time_eval():
                idx = np.asarray(
                    jax.random.randint(jax.random.key(seed), (_L, _U), 0, _L))
            np.add.at(w[c], (np.arange(_L)[:, None], idx), 1.0)
        bias = np.where(w > 0, np.float32(0), np.float32(-np.inf))
        _W_CACHE.append(w.astype(ml_dtypes.bfloat16))
        _W_CACHE.append(bias.astype(ml_dtypes.bfloat16))
        return jnp.asarray(_W_CACHE[0]), jnp.asarray(_W_CACHE[1])
    except Exception:
        ws = []
        rows = jnp.arange(_L)[:, None]
        for seed in (42, 43):
            idx = jax.random.randint(jax.random.key(seed), (_L, _U), 0, _L)
            ws.append(jnp.zeros((_L, _L), jnp.float32).at[rows, idx].add(1.0))
        w = jnp.stack(ws)
        bias = jnp.where(w > 0, jnp.float32(0), jnp.float32(-jnp.inf))
        return w.astype(jnp.bfloat16), bias.astype(jnp.bfloat16)


def _ln(x, g, b):
    m = jnp.mean(x, axis=1, keepdims=True)
    v = jnp.mean((x - m) ** 2, axis=1, keepdims=True)
    return (x - m) / jnp.sqrt(v + _EPS) * g + b


def _ln_qkv_kernel(x_ref, g_ref, b_ref, w_ref, bias_ref, qkv_ref, xn_ref):
    xn = _ln(x_ref[0], g_ref[...], b_ref[...])
    xn_ref[0] = xn
    qkv_ref[0] = (
        jnp.dot(xn, w_ref[...], preferred_element_type=jnp.float32) + bias_ref[...]
    )


def _m_scores_kernel(q_ref, k_ref, w_ref, bias_ref, m_ref):
    kw = jnp.dot(w_ref[0], k_ref[0].astype(jnp.bfloat16),
                 preferred_element_type=jnp.float32)
    bias = bias_ref[0]
    for h in range(_H):
        q = q_ref[0, :, h * _E:(h + 1) * _E]
        k = k_ref[0, :, h * _E:(h + 1) * _E]
        s = jax.lax.dot_general(
            q, k, (((1,), (1,)), ((), ())), preferred_element_type=jnp.float32
        )
        mx = jnp.max(s + bias, axis=1)
        sm = jnp.sum(q * kw[:, h * _E:(h + 1) * _E], axis=1)
        m_ref[0, 0, h, :] = mx - sm * (1.0 / _L)


_HG = 8
_NBH = 2 * _B * _H


def _topk_kernel(m_ref, sel_ref):
    m = m_ref[...].reshape(_NBH, _L)
    iota_l = jax.lax.broadcasted_iota(jnp.int32, (_NBH, _L), 1)
    iota_u = jax.lax.broadcasted_iota(jnp.int32, (_NBH, _U), 1)

    def body(i, carry):
        m, sel = carry
        rowmax = jnp.max(m, axis=1, keepdims=True)
        first = jnp.min(
            jnp.where(m == rowmax, iota_l, _L), axis=1, keepdims=True
        )
        sel = jnp.where(iota_u == i, first, sel)
        m = jnp.where(iota_l == first, -jnp.inf, m)
        return m, sel

    _, sel = jax.lax.fori_loop(
        0, _U, body, (m, jnp.zeros((_NBH, _U), jnp.int32))
    )
    sel_ref[...] = sel.reshape(2, _B, _H, _U)


def _attn_kernel(sel_ref, q_ref, k_ref, v_ref, out_ref):
    sel = sel_ref[0, 0]
    onehot_iota = jax.lax.broadcasted_iota(jnp.int32, (_U, _L), 1)
    for h in range(_HG):
        q = q_ref[0, :, h * _E:(h + 1) * _E]
        k = k_ref[0, :, h * _E:(h + 1) * _E]
        v = v_ref[0, :, h * _E:(h + 1) * _E]
        oh = (onehot_iota == sel[h][:, None]).astype(jnp.float32)
        qred = jnp.dot(oh, q, preferred_element_type=jnp.float32)
        s = jax.lax.dot_general(
            qred, k, (((1,), (1,)), ((), ())), preferred_element_type=jnp.float32
        ) * _SCALE
        s = s - jnp.max(s, axis=1, keepdims=True)
        e = jnp.exp(s)
        attn = e / jnp.sum(e, axis=1, keepdims=True)
        upd = jnp.dot(attn, v, preferred_element_type=jnp.float32)
        mean_v = jnp.mean(v, axis=0, keepdims=True)
        scat = jax.lax.dot_general(
            oh, upd, (((0,), (0,)), ((), ())), preferred_element_type=jnp.float32
        )
        ctx = scat + (1.0 - jnp.sum(oh, axis=0))[:, None] * mean_v
        out_ref[0, 0, :, h * _E:(h + 1) * _E] = ctx.astype(jnp.bfloat16)


def _wo_ln_kernel(ctx_ref, xn_ref, w_ref, b_ref, g_ref, bb_ref, y_ref):
    t = (
        jnp.dot(ctx_ref[0], w_ref[...],
                preferred_element_type=jnp.float32)
        + b_ref[...]
        + xn_ref[0]
    )
    y_ref[0] = _ln(t, g_ref[...], bb_ref[...])


def _mlp_kernel(y_ref, w1_ref, b1_ref, g1_ref, bb1_ref, w2_ref, b2_ref,
                g2_ref, bb2_ref, o_ref):
    y = y_ref[0]
    t = (
        jnp.dot(y.astype(jnp.bfloat16), w1_ref[...],
                preferred_element_type=jnp.float32)
        + b1_ref[...]
    )
    t = _ln(t, g1_ref[...], bb1_ref[...])
    t = t * 0.5 * (1.0 + jax.lax.erf(t * np.float32(1.0 / np.sqrt(2.0))))
    t = (
        jnp.dot(t.astype(jnp.bfloat16), w2_ref[...],
                preferred_element_type=jnp.float32)
        + b2_ref[...]
    )
    o_ref[0] = y + _ln(t, g2_ref[...], bb2_ref[...])


def kernel(input, ln1_g, ln1_b, ln2_g, ln2_b, Wq, bq, Wk, bk, Wv, bv, Wo, bo,
           fc1_W, fc1_b, mln1_g, mln1_b, fc2_W, fc2_b, mln2_g, mln2_b):
    f32 = jnp.float32
    x = input.reshape(2, _NR, _DIM)
    wqkv = jnp.concatenate([Wq, Wk, Wv], axis=1)
    bqkv = jnp.concatenate([bq, bk, bv])[None, :]
    w_counts, w_bias = _sample_counts()

    qkv, xn = pl.pallas_call(
        _ln_qkv_kernel,
        grid=(2, _NR // _RT),
        in_specs=[
            pl.BlockSpec((1, _RT, _DIM), lambda c, r: (c, r, 0)),
            pl.BlockSpec((1, _DIM), lambda c, r: (0, 0)),
            pl.BlockSpec((1, _DIM), lambda c, r: (0, 0)),
            pl.BlockSpec((_DIM, 3 * _DIM), lambda c, r: (0, 0)),
            pl.BlockSpec((1, 3 * _DIM), lambda c, r: (0, 0)),
        ],
        out_specs=[
            pl.BlockSpec((1, _RT, 3 * _DIM), lambda c, r: (c, r, 0)),
            pl.BlockSpec((1, _RT, _DIM), lambda c, r: (c, r, 0)),
        ],
        out_shape=[
            jax.ShapeDtypeStruct((2, _NR, 3 * _DIM), f32),
            jax.ShapeDtypeStruct((2, _NR, _DIM), f32),
        ],
    )(x, ln1_g[None, :], ln1_b[None, :], wqkv, bqkv)

    nqt = _L // _QT

    m_arr = pl.pallas_call(
        _m_scores_kernel,
        grid=(2, _B, nqt),
        in_specs=[
            pl.BlockSpec((1, _QT, _DIM),
                         lambda c, b, qt: (1 - c, b * (_L // _QT) + qt, 0)),
            pl.BlockSpec((1, _L, _DIM), lambda c, b, qt: (c, b, 1)),
            pl.BlockSpec((1, _QT, _L), lambda c, b, qt: (c, qt, 0)),
            pl.BlockSpec((1, _QT, _L), lambda c, b, qt: (c, qt, 0)),
        ],
        out_specs=pl.BlockSpec((1, 1, _H, _QT), lambda c, b, qt: (c, b, 0, qt)),
        out_shape=jax.ShapeDtypeStruct((2, _B, _H, _L), f32),
    )(qkv, qkv, w_counts, w_bias)

    sel = pl.pallas_call(
        _topk_kernel,
        grid=(1,),
        in_specs=[pl.BlockSpec((2, _B, _H, _L), lambda i: (0, 0, 0, 0))],
        out_specs=pl.BlockSpec((2, _B, _H, _U), lambda i: (0, 0, 0, 0)),
        out_shape=jax.ShapeDtypeStruct((2, _B, _H, _U), jnp.int32),
    )(m_arr)

    hgw = _HG * _E
    ctx = pl.pallas_call(
        _attn_kernel,
        grid=(2, _B, _H // _HG),
        in_specs=[
            pl.BlockSpec((1, 1, _HG, _U), lambda c, b, g: (c, b, g, 0)),
            pl.BlockSpec((1, _L, hgw),
                         lambda c, b, g: (1 - c, b, g)),
            pl.BlockSpec((1, _L, hgw),
                         lambda c, b, g: (c, b, _DIM // hgw + g)),
            pl.BlockSpec((1, _L, hgw),
                         lambda c, b, g: (c, b, 2 * (_DIM // hgw) + g)),
        ],
        out_specs=pl.BlockSpec((1, 1, _L, hgw), lambda c, b, g: (c, b, 0, g)),
        out_shape=jax.ShapeDtypeStruct((2, _B, _L, _DIM), jnp.bfloat16),
    )(sel, qkv, qkv, qkv)

    y = pl.pallas_call(
        _wo_ln_kernel,
        grid=(2, _NR // _RT),
        in_specs=[
            pl.BlockSpec((1, _RT, _DIM), lambda c, r: (c, r, 0)),
            pl.BlockSpec((1, _RT, _DIM), lambda c, r: (c, r, 0)),
            pl.BlockSpec((_DIM, _DIM), lambda c, r: (0, 0)),
            pl.BlockSpec((1, _DIM), lambda c, r: (0, 0)),
            pl.BlockSpec((1, _DIM), lambda c, r: (0, 0)),
            pl.BlockSpec((1, _DIM), lambda c, r: (0, 0)),
        ],
        out_specs=pl.BlockSpec((1, _RT, _DIM), lambda c, r: (c, r, 0)),
        out_shape=jax.ShapeDtypeStruct((2, _NR, _DIM), f32),
    )(ctx.reshape(2, _NR, _DIM), xn, Wo.astype(jnp.bfloat16), bo[None, :],
      ln2_g[None, :], ln2_b[None, :])

    out = pl.pallas_call(
        _mlp_kernel,
        grid=(2, _NR // _RTM),
        in_specs=[
            pl.BlockSpec((1, _RTM, _DIM), lambda c, r: (c, r, 0)),
            pl.BlockSpec((_DIM, _HID), lambda c, r: (0, 0)),
            pl.BlockSpec((1, _HID), lambda c, r: (0, 0)),
            pl.BlockSpec((1, _HID), lambda c, r: (0, 0)),
            pl.BlockSpec((1, _HID), lambda c, r: (0, 0)),
            pl.BlockSpec((_HID, _DIM), lambda c, r: (0, 0)),
            pl.BlockSpec((1, _DIM), lambda c, r: (0, 0)),
            pl.BlockSpec((1, _DIM), lambda c, r: (0, 0)),
            pl.BlockSpec((1, _DIM), lambda c, r: (0, 0)),
        ],
        out_specs=pl.BlockSpec((1, _RTM, _DIM), lambda c, r: (c, r, 0)),
        out_shape=jax.ShapeDtypeStruct((2, _NR, _DIM), f32),
    )(y, fc1_W.astype(jnp.bfloat16), fc1_b[None, :], mln1_g[None, :],
      mln1_b[None, :], fc2_W.astype(jnp.bfloat16), fc2_b[None, :],
      mln2_g[None, :], mln2_b[None, :])

    out = out.reshape(2, _B, _L, _DIM)
    return (out[0], out[1])


# speedup vs baseline: 1.8005x; 1.0713x over previous
import math

import jax
import jax.numpy as jnp
import numpy as np
from jax.experimental import pallas as pl

_DIM = 1024
_H = 16
_E = 64
_HID = 4096
_B = 2
_L = 2048
_NR = _B * _L
_U = 40
_EPS = 1e-5
_SCALE = 1.0 / math.sqrt(_E)
_QT = 256
_RT = 512
_RTM = 256


_W_CACHE = []


def _sample_counts():
    if _W_CACHE:
        return jnp.asarray(_W_CACHE[0]), jnp.asarray(_W_CACHE[1])
    try:
        import ml_dtypes
        w = np.zeros((2, _L, _L), np.float32)
        for c, seed in enumerate((42, 43)):
            with jax.ensure_compile_time_eval():
                idx = np.asarray(
                    jax.random.randint(jax.random.key(seed), (_L, _U), 0, _L))
            np.add.at(w[c], (np.arange(_L)[:, None], idx), 1.0)
        bias = np.where(w > 0, np.float32(0), np.float32(-np.inf))
        _W_CACHE.append(w.astype(ml_dtypes.bfloat16))
        _W_CACHE.append(bias.astype(ml_dtypes.bfloat16))
        return jnp.asarray(_W_CACHE[0]), jnp.asarray(_W_CACHE[1])
    except Exception:
        ws = []
        rows = jnp.arange(_L)[:, None]
        for seed in (42, 43):
            idx = jax.random.randint(jax.random.key(seed), (_L, _U), 0, _L)
            ws.append(jnp.zeros((_L, _L), jnp.float32).at[rows, idx].add(1.0))
        w = jnp.stack(ws)
        bias = jnp.where(w > 0, jnp.float32(0), jnp.float32(-jnp.inf))
        return w.astype(jnp.bfloat16), bias.astype(jnp.bfloat16)


def _ln(x, g, b):
    m = jnp.mean(x, axis=1, keepdims=True)
    v = jnp.mean((x - m) ** 2, axis=1, keepdims=True)
    return (x - m) / jnp.sqrt(v + _EPS) * g + b


def _ln_qkv_kernel(x_ref, g_ref, b_ref, w_ref, bias_ref, qkv_ref, xn_ref):
    xn = _ln(x_ref[0], g_ref[...], b_ref[...])
    xn_ref[0] = xn
    qkv_ref[0] = (
        jnp.dot(xn, w_ref[...], preferred_element_type=jnp.float32) + bias_ref[...]
    )


def _m_scores_kernel(q_ref, k_ref, w_ref, bias_ref, m_ref):
    kb = k_ref[0].astype(jnp.bfloat16)
    qb = q_ref[0].astype(jnp.bfloat16)
    kw = jnp.dot(w_ref[0], kb, preferred_element_type=jnp.float32)
    bias = bias_ref[0]
    for h in range(_H):
        q = q_ref[0, :, h * _E:(h + 1) * _E]
        s = jax.lax.dot_general(
            qb[:, h * _E:(h + 1) * _E], kb[:, h * _E:(h + 1) * _E],
            (((1,), (1,)), ((), ())), preferred_element_type=jnp.float32
        )
        mx = jnp.max(s + bias, axis=1)
        sm = jnp.sum(q * kw[:, h * _E:(h + 1) * _E], axis=1)
        m_ref[0, 0, h, :] = mx - sm * (1.0 / _L)


_HG = 8
_NBH = 2 * _B * _H


def _topk_kernel(m_ref, sel_ref):
    m = m_ref[...].reshape(_NBH, _L)
    iota_l = jax.lax.broadcasted_iota(jnp.int32, (_NBH, _L), 1)
    iota_u = jax.lax.broadcasted_iota(jnp.int32, (_NBH, _U), 1)

    def body(i, carry):
        m, sel = carry
        rowmax = jnp.max(m, axis=1, keepdims=True)
        first = jnp.min(
            jnp.where(m == rowmax, iota_l, _L), axis=1, keepdims=True
        )
        sel = jnp.where(iota_u == i, first, sel)
        m = jnp.where(iota_l == first, -jnp.inf, m)
        return m, sel

    _, sel = jax.lax.fori_loop(
        0, _U, body, (m, jnp.zeros((_NBH, _U), jnp.int32))
    )
    sel_ref[...] = sel.reshape(2, _B, _H, _U)


def _attn_kernel(sel_ref, q_ref, k_ref, v_ref, out_ref):
    sel = sel_ref[0, 0]
    onehot_iota = jax.lax.broadcasted_iota(jnp.int32, (_U, _L), 1)
    qall = q_ref[0].astype(jnp.bfloat16)
    kall = k_ref[0].astype(jnp.bfloat16)
    vall = v_ref[0].astype(jnp.bfloat16)
    for h in range(_HG):
        q = qall[:, h * _E:(h + 1) * _E]
        k = kall[:, h * _E:(h + 1) * _E]
        v = vall[:, h * _E:(h + 1) * _E]
        oh = (onehot_iota == sel[h][:, None]).astype(jnp.bfloat16)
        qred = jnp.dot(oh, q, preferred_element_type=jnp.float32)
        s = jax.lax.dot_general(
            qred.astype(jnp.bfloat16), k,
            (((1,), (1,)), ((), ())), preferred_element_type=jnp.float32
        ) * _SCALE
        s = s - jnp.max(s, axis=1, keepdims=True)
        e = jnp.exp(s)
        attn = e / jnp.sum(e, axis=1, keepdims=True)
        upd = jnp.dot(attn.astype(jnp.bfloat16), v,
                      preferred_element_type=jnp.float32)
        mean_v = jnp.mean(v.astype(jnp.float32), axis=0, keepdims=True)
        scat = jax.lax.dot_general(
            oh, upd.astype(jnp.bfloat16),
            (((0,), (0,)), ((), ())), preferred_element_type=jnp.float32
        )
        ctx = scat + (1.0 - jnp.sum(oh, axis=0))[:, None] * mean_v
        out_ref[0, 0, :, h * _E:(h + 1) * _E] = ctx.astype(jnp.bfloat16)


def _wo_ln_kernel(ctx_ref, xn_ref, w_ref, b_ref, g_ref, bb_ref, y_ref):
    t = (
        jnp.dot(ctx_ref[0], w_ref[...],
                preferred_element_type=jnp.float32)
        + b_ref[...]
        + xn_ref[0]
    )
    y_ref[0] = _ln(t, g_ref[...], bb_ref[...])


def _mlp_kernel(y_ref, w1_ref, b1_ref, g1_ref, bb1_ref, w2_ref, b2_ref,
                g2_ref, bb2_ref, o_ref):
    y = y_ref[0]
    t = (
        jnp.dot(y.astype(jnp.bfloat16), w1_ref[...],
                preferred_element_type=jnp.float32)
        + b1_ref[...]
    )
    t = _ln(t, g1_ref[...], bb1_ref[...])
    t = t * 0.5 * (1.0 + jax.lax.erf(t * np.float32(1.0 / np.sqrt(2.0))))
    t = (
        jnp.dot(t.astype(jnp.bfloat16), w2_ref[...],
                preferred_element_type=jnp.float32)
        + b2_ref[...]
    )
    o_ref[0] = y + _ln(t, g2_ref[...], bb2_ref[...])


def kernel(input, ln1_g, ln1_b, ln2_g, ln2_b, Wq, bq, Wk, bk, Wv, bv, Wo, bo,
           fc1_W, fc1_b, mln1_g, mln1_b, fc2_W, fc2_b, mln2_g, mln2_b):
    f32 = jnp.float32
    x = input.reshape(2, _NR, _DIM)
    wqkv = jnp.concatenate([Wq, Wk, Wv], axis=1)
    bqkv = jnp.concatenate([bq, bk, bv])[None, :]
    w_counts, w_bias = _sample_counts()

    qkv, xn = pl.pallas_call(
        _ln_qkv_kernel,
        grid=(2, _NR // _RT),
        in_specs=[
            pl.BlockSpec((1, _RT, _DIM), lambda c, r: (c, r, 0)),
            pl.BlockSpec((1, _DIM), lambda c, r: (0, 0)),
            pl.BlockSpec((1, _DIM), lambda c, r: (0, 0)),
            pl.BlockSpec((_DIM, 3 * _DIM), lambda c, r: (0, 0)),
            pl.BlockSpec((1, 3 * _DIM), lambda c, r: (0, 0)),
        ],
        out_specs=[
            pl.BlockSpec((1, _RT, 3 * _DIM), lambda c, r: (c, r, 0)),
            pl.BlockSpec((1, _RT, _DIM), lambda c, r: (c, r, 0)),
        ],
        out_shape=[
            jax.ShapeDtypeStruct((2, _NR, 3 * _DIM), f32),
            jax.ShapeDtypeStruct((2, _NR, _DIM), f32),
        ],
    )(x, ln1_g[None, :], ln1_b[None, :], wqkv, bqkv)

    nqt = _L // _QT

    m_arr = pl.pallas_call(
        _m_scores_kernel,
        grid=(2, _B, nqt),
        in_specs=[
            pl.BlockSpec((1, _QT, _DIM),
                         lambda c, b, qt: (1 - c, b * (_L // _QT) + qt, 0)),
            pl.BlockSpec((1, _L, _DIM), lambda c, b, qt: (c, b, 1)),
            pl.BlockSpec((1, _QT, _L), lambda c, b, qt: (c, qt, 0)),
            pl.BlockSpec((1, _QT, _L), lambda c, b, qt: (c, qt, 0)),
        ],
        out_specs=pl.BlockSpec((1, 1, _H, _QT), lambda c, b, qt: (c, b, 0, qt)),
        out_shape=jax.ShapeDtypeStruct((2, _B, _H, _L), f32),
    )(qkv, qkv, w_counts, w_bias)

    sel = pl.pallas_call(
        _topk_kernel,
        grid=(1,),
        in_specs=[pl.BlockSpec((2, _B, _H, _L), lambda i: (0, 0, 0, 0))],
        out_specs=pl.BlockSpec((2, _B, _H, _U), lambda i: (0, 0, 0, 0)),
        out_shape=jax.ShapeDtypeStruct((2, _B, _H, _U), jnp.int32),
    )(m_arr)

    hgw = _HG * _E
    ctx = pl.pallas_call(
        _attn_kernel,
        grid=(2, _B, _H // _HG),
        in_specs=[
            pl.BlockSpec((1, 1, _HG, _U), lambda c, b, g: (c, b, g, 0)),
            pl.BlockSpec((1, _L, hgw),
                         lambda c, b, g: (1 - c, b, g)),
            pl.BlockSpec((1, _L, hgw),
                         lambda c, b, g: (c, b, _DIM // hgw + g)),
            pl.BlockSpec((1, _L, hgw),
                         lambda c, b, g: (c, b, 2 * (_DIM // hgw) + g)),
        ],
        out_specs=pl.BlockSpec((1, 1, _L, hgw), lambda c, b, g: (c, b, 0, g)),
        out_shape=jax.ShapeDtypeStruct((2, _B, _L, _DIM), jnp.bfloat16),
    )(sel, qkv, qkv, qkv)

    y = pl.pallas_call(
        _wo_ln_kernel,
        grid=(2, _NR // _RT),
        in_specs=[
            pl.BlockSpec((1, _RT, _DIM), lambda c, r: (c, r, 0)),
            pl.BlockSpec((1, _RT, _DIM), lambda c, r: (c, r, 0)),
            pl.BlockSpec((_DIM, _DIM), lambda c, r: (0, 0)),
            pl.BlockSpec((1, _DIM), lambda c, r: (0, 0)),
            pl.BlockSpec((1, _DIM), lambda c, r: (0, 0)),
            pl.BlockSpec((1, _DIM), lambda c, r: (0, 0)),
        ],
        out_specs=pl.BlockSpec((1, _RT, _DIM), lambda c, r: (c, r, 0)),
        out_shape=jax.ShapeDtypeStruct((2, _NR, _DIM), f32),
    )(ctx.reshape(2, _NR, _DIM), xn, Wo.astype(jnp.bfloat16), bo[None, :],
      ln2_g[None, :], ln2_b[None, :])

    out = pl.pallas_call(
        _mlp_kernel,
        grid=(2, _NR // _RTM),
        in_specs=[
            pl.BlockSpec((1, _RTM, _DIM), lambda c, r: (c, r, 0)),
            pl.BlockSpec((_DIM, _HID), lambda c, r: (0, 0)),
            pl.BlockSpec((1, _HID), lambda c, r: (0, 0)),
            pl.BlockSpec((1, _HID), lambda c, r: (0, 0)),
            pl.BlockSpec((1, _HID), lambda c, r: (0, 0)),
            pl.BlockSpec((_HID, _DIM), lambda c, r: (0, 0)),
            pl.BlockSpec((1, _DIM), lambda c, r: (0, 0)),
            pl.BlockSpec((1, _DIM), lambda c, r: (0, 0)),
            pl.BlockSpec((1, _DIM), lambda c, r: (0, 0)),
        ],
        out_specs=pl.BlockSpec((1, _RTM, _DIM), lambda c, r: (c, r, 0)),
        out_shape=jax.ShapeDtypeStruct((2, _NR, _DIM), f32),
    )(y, fc1_W.astype(jnp.bfloat16), fc1_b[None, :], mln1_g[None, :],
      mln1_b[None, :], fc2_W.astype(jnp.bfloat16), fc2_b[None, :],
      mln2_g[None, :], mln2_b[None, :])

    out = out.reshape(2, _B, _L, _DIM)
    return (out[0], out[1])


# speedup vs baseline: 1.8278x; 1.0152x over previous
import math

import jax
import jax.numpy as jnp
import numpy as np
from jax.experimental import pallas as pl

_DIM = 1024
_H = 16
_E = 64
_HID = 4096
_B = 2
_L = 2048
_NR = _B * _L
_U = 40
_EPS = 1e-5
_SCALE = 1.0 / math.sqrt(_E)
_QT = 256
_RT = 512
_RTM = 256


_W_CACHE = []


def _sample_counts():
    if _W_CACHE:
        return jnp.asarray(_W_CACHE[0]), jnp.asarray(_W_CACHE[1])
    try:
        import ml_dtypes
        w = np.zeros((2, _L, _L), np.float32)
        for c, seed in enumerate((42, 43)):
            with jax.ensure_compile_time_eval():
                idx = np.asarray(
                    jax.random.randint(jax.random.key(seed), (_L, _U), 0, _L))
            np.add.at(w[c], (np.arange(_L)[:, None], idx), 1.0)
        bias = np.where(w > 0, np.float32(0), np.float32(-np.inf))
        _W_CACHE.append(w.astype(ml_dtypes.bfloat16))
        _W_CACHE.append(bias.astype(ml_dtypes.bfloat16))
        return jnp.asarray(_W_CACHE[0]), jnp.asarray(_W_CACHE[1])
    except Exception:
        ws = []
        rows = jnp.arange(_L)[:, None]
        for seed in (42, 43):
            idx = jax.random.randint(jax.random.key(seed), (_L, _U), 0, _L)
            ws.append(jnp.zeros((_L, _L), jnp.float32).at[rows, idx].add(1.0))
        w = jnp.stack(ws)
        bias = jnp.where(w > 0, jnp.float32(0), jnp.float32(-jnp.inf))
        return w.astype(jnp.bfloat16), bias.astype(jnp.bfloat16)


def _ln(x, g, b):
    m = jnp.mean(x, axis=1, keepdims=True)
    v = jnp.mean((x - m) ** 2, axis=1, keepdims=True)
    return (x - m) / jnp.sqrt(v + _EPS) * g + b


def _ln_qkv_kernel(x_ref, g_ref, b_ref, w_ref, bias_ref, qkv_ref, xn_ref):
    xn = _ln(x_ref[0], g_ref[...], b_ref[...])
    xn_ref[0] = xn
    qkv_ref[0] = (
        jnp.dot(xn.astype(jnp.bfloat16), w_ref[...],
                preferred_element_type=jnp.float32) + bias_ref[...]
    ).astype(jnp.bfloat16)


def _m_scores_kernel(q_ref, k_ref, w_ref, bias_ref, m_ref):
    kb = k_ref[0]
    qb = q_ref[0]
    kw = jnp.dot(w_ref[0], kb, preferred_element_type=jnp.float32)
    bias = bias_ref[0]
    for h in range(_H):
        q = qb[:, h * _E:(h + 1) * _E].astype(jnp.float32)
        s = jax.lax.dot_general(
            qb[:, h * _E:(h + 1) * _E], kb[:, h * _E:(h + 1) * _E],
            (((1,), (1,)), ((), ())), preferred_element_type=jnp.float32
        )
        mx = jnp.max(s + bias, axis=1)
        sm = jnp.sum(q * kw[:, h * _E:(h + 1) * _E], axis=1)
        m_ref[0, 0, h, :] = mx - sm * (1.0 / _L)


_HG = 8
_NBH = 2 * _B * _H


def _topk_kernel(m_ref, sel_ref):
    m = m_ref[...].reshape(_NBH, _L)
    iota_l = jax.lax.broadcasted_iota(jnp.int32, (_NBH, _L), 1)
    iota_u = jax.lax.broadcasted_iota(jnp.int32, (_NBH, _U), 1)

    def body(i, carry):
        m, sel = carry
        rowmax = jnp.max(m, axis=1, keepdims=True)
        first = jnp.min(
            jnp.where(m == rowmax, iota_l, _L), axis=1, keepdims=True
        )
        sel = jnp.where(iota_u == i, first, sel)
        m = jnp.where(iota_l == first, -jnp.inf, m)
        return m, sel

    _, sel = jax.lax.fori_loop(
        0, _U, body, (m, jnp.zeros((_NBH, _U), jnp.int32))
    )
    sel_ref[...] = sel.reshape(2, _B, _H, _U)


def _attn_kernel(sel_ref, q_ref, k_ref, v_ref, out_ref):
    sel = sel_ref[0, 0]
    onehot_iota = jax.lax.broadcasted_iota(jnp.int32, (_U, _L), 1)
    qall = q_ref[0]
    kall = k_ref[0]
    vall = v_ref[0]
    for h in range(_HG):
        q = qall[:, h * _E:(h + 1) * _E]
        k = kall[:, h * _E:(h + 1) * _E]
        v = vall[:, h * _E:(h + 1) * _E]
        oh = (onehot_iota == sel[h][:, None]).astype(jnp.bfloat16)
        qred = jnp.dot(oh, q, preferred_element_type=jnp.float32)
        s = jax.lax.dot_general(
            qred.astype(jnp.bfloat16), k,
            (((1,), (1,)), ((), ())), preferred_element_type=jnp.float32
        ) * _SCALE
        s = s - jnp.max(s, axis=1, keepdims=True)
        e = jnp.exp(s)
        attn = e / jnp.sum(e, axis=1, keepdims=True)
        upd = jnp.dot(attn.astype(jnp.bfloat16), v,
                      preferred_element_type=jnp.float32)
        mean_v = jnp.mean(v.astype(jnp.float32), axis=0, keepdims=True)
        scat = jax.lax.dot_general(
            oh, upd.astype(jnp.bfloat16),
            (((0,), (0,)), ((), ())), preferred_element_type=jnp.float32
        )
        ctx = scat + (1.0 - jnp.sum(oh, axis=0))[:, None] * mean_v
        out_ref[0, 0, :, h * _E:(h + 1) * _E] = ctx.astype(jnp.bfloat16)


def _wo_ln_kernel(ctx_ref, xn_ref, w_ref, b_ref, g_ref, bb_ref, y_ref):
    t = (
        jnp.dot(ctx_ref[0], w_ref[...],
                preferred_element_type=jnp.float32)
        + b_ref[...]
        + xn_ref[0]
    )
    y_ref[0] = _ln(t, g_ref[...], bb_ref[...]).astype(jnp.bfloat16)


def _mlp_kernel(y_ref, w1_ref, b1_ref, g1_ref, bb1_ref, w2_ref, b2_ref,
                g2_ref, bb2_ref, o_ref):
    y = y_ref[0]
    t = (
        jnp.dot(y, w1_ref[...], preferred_element_type=jnp.float32)
        + b1_ref[...]
    )
    t = _ln(t, g1_ref[...], bb1_ref[...])
    t = t * 0.5 * (1.0 + jax.lax.erf(t * np.float32(1.0 / np.sqrt(2.0))))
    t = (
        jnp.dot(t.astype(jnp.bfloat16), w2_ref[...],
                preferred_element_type=jnp.float32)
        + b2_ref[...]
    )
    o_ref[0] = y.astype(jnp.float32) + _ln(t, g2_ref[...], bb2_ref[...])


def kernel(input, ln1_g, ln1_b, ln2_g, ln2_b, Wq, bq, Wk, bk, Wv, bv, Wo, bo,
           fc1_W, fc1_b, mln1_g, mln1_b, fc2_W, fc2_b, mln2_g, mln2_b):
    f32 = jnp.float32
    x = input.reshape(2, _NR, _DIM)
    wqkv = jnp.concatenate([Wq, Wk, Wv], axis=1).astype(jnp.bfloat16)
    bqkv = jnp.concatenate([bq, bk, bv])[None, :]
    w_counts, w_bias = _sample_counts()

    qkv, xn = pl.pallas_call(
        _ln_qkv_kernel,
        grid=(2, _NR // _RT),
        in_specs=[
            pl.BlockSpec((1, _RT, _DIM), lambda c, r: (c, r, 0)),
            pl.BlockSpec((1, _DIM), lambda c, r: (0, 0)),
            pl.BlockSpec((1, _DIM), lambda c, r: (0, 0)),
            pl.BlockSpec((_DIM, 3 * _DIM), lambda c, r: (0, 0)),
            pl.BlockSpec((1, 3 * _DIM), lambda c, r: (0, 0)),
        ],
        out_specs=[
            pl.BlockSpec((1, _RT, 3 * _DIM), lambda c, r: (c, r, 0)),
            pl.BlockSpec((1, _RT, _DIM), lambda c, r: (c, r, 0)),
        ],
        out_shape=[
            jax.ShapeDtypeStruct((2, _NR, 3 * _DIM), jnp.bfloat16),
            jax.ShapeDtypeStruct((2, _NR, _DIM), f32),
        ],
    )(x, ln1_g[None, :], ln1_b[None, :], wqkv, bqkv)

    nqt = _L // _QT

    m_arr = pl.pallas_call(
        _m_scores_kernel,
        grid=(2, _B, nqt),
        in_specs=[
            pl.BlockSpec((1, _QT, _DIM),
                         lambda c, b, qt: (1 - c, b * (_L // _QT) + qt, 0)),
            pl.BlockSpec((1, _L, _DIM), lambda c, b, qt: (c, b, 1)),
            pl.BlockSpec((1, _QT, _L), lambda c, b, qt: (c, qt, 0)),
            pl.BlockSpec((1, _QT, _L), lambda c, b, qt: (c, qt, 0)),
        ],
        out_specs=pl.BlockSpec((1, 1, _H, _QT), lambda c, b, qt: (c, b, 0, qt)),
        out_shape=jax.ShapeDtypeStruct((2, _B, _H, _L), f32),
    )(qkv, qkv, w_counts, w_bias)

    sel = pl.pallas_call(
        _topk_kernel,
        grid=(1,),
        in_specs=[pl.BlockSpec((2, _B, _H, _L), lambda i: (0, 0, 0, 0))],
        out_specs=pl.BlockSpec((2, _B, _H, _U), lambda i: (0, 0, 0, 0)),
        out_shape=jax.ShapeDtypeStruct((2, _B, _H, _U), jnp.int32),
    )(m_arr)

    hgw = _HG * _E
    ctx = pl.pallas_call(
        _attn_kernel,
        grid=(2, _B, _H // _HG),
        in_specs=[
            pl.BlockSpec((1, 1, _HG, _U), lambda c, b, g: (c, b, g, 0)),
            pl.BlockSpec((1, _L, hgw),
                         lambda c, b, g: (1 - c, b, g)),
            pl.BlockSpec((1, _L, hgw),
                         lambda c, b, g: (c, b, _DIM // hgw + g)),
            pl.BlockSpec((1, _L, hgw),
                         lambda c, b, g: (c, b, 2 * (_DIM // hgw) + g)),
        ],
        out_specs=pl.BlockSpec((1, 1, _L, hgw), lambda c, b, g: (c, b, 0, g)),
        out_shape=jax.ShapeDtypeStruct((2, _B, _L, _DIM), jnp.bfloat16),
    )(sel, qkv, qkv, qkv)

    y = pl.pallas_call(
        _wo_ln_kernel,
        grid=(2, _NR // _RT),
        in_specs=[
            pl.BlockSpec((1, _RT, _DIM), lambda c, r: (c, r, 0)),
            pl.BlockSpec((1, _RT, _DIM), lambda c, r: (c, r, 0)),
            pl.BlockSpec((_DIM, _DIM), lambda c, r: (0, 0)),
            pl.BlockSpec((1, _DIM), lambda c, r: (0, 0)),
            pl.BlockSpec((1, _DIM), lambda c, r: (0, 0)),
            pl.BlockSpec((1, _DIM), lambda c, r: (0, 0)),
        ],
        out_specs=pl.BlockSpec((1, _RT, _DIM), lambda c, r: (c, r, 0)),
        out_shape=jax.ShapeDtypeStruct((2, _NR, _DIM), jnp.bfloat16),
    )(ctx.reshape(2, _NR, _DIM), xn, Wo.astype(jnp.bfloat16), bo[None, :],
      ln2_g[None, :], ln2_b[None, :])

    out = pl.pallas_call(
        _mlp_kernel,
        grid=(2, _NR // _RTM),
        in_specs=[
            pl.BlockSpec((1, _RTM, _DIM), lambda c, r: (c, r, 0)),
            pl.BlockSpec((_DIM, _HID), lambda c, r: (0, 0)),
            pl.BlockSpec((1, _HID), lambda c, r: (0, 0)),
            pl.BlockSpec((1, _HID), lambda c, r: (0, 0)),
            pl.BlockSpec((1, _HID), lambda c, r: (0, 0)),
            pl.BlockSpec((_HID, _DIM), lambda c, r: (0, 0)),
            pl.BlockSpec((1, _DIM), lambda c, r: (0, 0)),
            pl.BlockSpec((1, _DIM), lambda c, r: (0, 0)),
            pl.BlockSpec((1, _DIM), lambda c, r: (0, 0)),
        ],
        out_specs=pl.BlockSpec((1, _RTM, _DIM), lambda c, r: (c, r, 0)),
        out_shape=jax.ShapeDtypeStruct((2, _NR, _DIM), f32),
    )(y, fc1_W.astype(jnp.bfloat16), fc1_b[None, :], mln1_g[None, :],
      mln1_b[None, :], fc2_W.astype(jnp.bfloat16), fc2_b[None, :],
      mln2_g[None, :], mln2_b[None, :])

    out = out.reshape(2, _B, _L, _DIM)
    return (out[0], out[1])
